# Initial kernel scaffold; baseline (speedup 1.0000x reference)
import jax
import jax.numpy as jnp
from jax.experimental import pallas as pl


def kernel(x_protein, x_term, edge_index_pp, edge_index_pt, edge_index_tp, W_in_p, b_in_p, W_in_t, b_in_t, Wl0_pp, bl0_pp, Wr0_pp, Wl0_pt, bl0_pt, Wr0_pt, Wl0_tp, bl0_tp, Wr0_tp, Wl1_pp, bl1_pp, Wr1_pp, Wl1_pt, bl1_pt, Wr1_pt, Wl1_tp, bl1_tp, Wr1_tp):
    raise NotImplementedError("write your pallas kernel here")


# speedup vs baseline: 2.0208x; 2.0208x over previous
import functools

import jax
import jax.numpy as jnp
from jax import lax
from jax.experimental import pallas as pl
from jax.experimental.pallas import tpu as pltpu
from jax.experimental.pallas import tpu_sc as plsc

_NP = 50000
_NT = 10000
_F = 128
_S = 16
_NSH = 8
_EB = 1024
_NR = _EB // 128
_DUMP = 64
_TILES = 16


def _pad_to(n, m):
    return ((n + m - 1) // m) * m


def _seg_sum_sc(x4, src1, dst2, n_src, n_dstp, with_counts):
    e_pad = src1.shape[0]
    rows_e_tile = e_pad // 128 // _TILES
    nblocks = rows_e_tile // _NR
    rpt = n_dstp // _TILES
    nz, rem = rpt // 128, rpt % 128
    n_out_sh = _NSH + (1 if with_counts else 0)
    mesh = plsc.VectorSubcoreMesh(core_axis_name="c", subcore_axis_name="s")

    @functools.partial(
        pl.kernel,
        mesh=mesh,
        compiler_params=pltpu.CompilerParams(use_tc_tiling_on_sc=False),
        out_type=jax.ShapeDtypeStruct((n_out_sh * n_dstp, _S), jnp.float32),
        scratch_types=[
            pltpu.VMEM((_EB,), jnp.int32),
            pltpu.VMEM((_NR, 128), jnp.int32),
            pltpu.VMEM((_EB, _S), jnp.float32),
            pltpu.VMEM((128, _S), jnp.float32),
            pltpu.VMEM_SHARED((n_dstp, _S), jnp.float32),
            pltpu.SemaphoreType.DMA,
        ],
    )
    def k(x4_hbm, src_hbm, dst_hbm, out_hbm, sidx, didx, rows, cbuf, acc, sem):
        c = lax.axis_index("c")
        t = lax.axis_index("s")
        mo8 = lambda v: pl.multiple_of(v, 8)

        def fill_cbuf(val):
            def body(i, carry):
                for h in range(_S // 16):
                    cbuf[i, pl.ds(h * 16, 16)] = jnp.full((16,), val, jnp.float32)
                return carry
            lax.fori_loop(0, 128, body, 0)

        def zero_acc():
            base = t * rpt
            def body(i, carry):
                pltpu.sync_copy(cbuf, acc.at[pl.ds(mo8(base + i * 128), 128)])
                return carry
            lax.fori_loop(0, nz, body, 0)
            if rem:
                pltpu.sync_copy(cbuf.at[pl.ds(0, rem)],
                                acc.at[pl.ds(mo8(base + nz * 128), rem)])

        def edge_pass(j_val, do_gather):
            e0_tile = t * (rows_e_tile * 128)
            off = (j_val * n_src).astype(jnp.int32)

            def blk(b, carry):
                e0 = mo8(e0_tile + b * _EB)
                r0 = mo8((e0_tile // 128) + b * _NR)
                pltpu.sync_copy(dst_hbm.at[pl.ds(r0, _NR)], didx)
                if do_gather:
                    pltpu.sync_copy(src_hbm.at[pl.ds(e0, _EB)], sidx)

                    def addoff(q, carry2):
                        v = sidx[pl.ds(q * 16, 16)]
                        sidx[pl.ds(q * 16, 16)] = v + off
                        return carry2
                    lax.fori_loop(0, _EB // 16, addoff, 0)
                    copies = []
                    for i in range(_NR):
                        copies.append(pltpu.async_copy(
                            x4_hbm.at[sidx.at[pl.ds(i * 128, 128)]],
                            rows.at[pl.ds(i * 128, 128)], sem))
                    for cp in copies:
                        cp.wait()
                    for i in range(_NR):
                        pltpu.sync_copy(rows.at[pl.ds(i * 128, 128)],
                                        acc.at[didx.at[i]], add=True)
                else:
                    for i in range(_NR):
                        pltpu.sync_copy(cbuf, acc.at[didx.at[i]], add=True)
                return carry
            lax.fori_loop(0, nblocks, blk, 0)

        def copy_out(j_val):
            src_base = t * rpt
            out_base = j_val * n_dstp + t * rpt
            def body(i, carry):
                pltpu.sync_copy(acc.at[pl.ds(mo8(src_base + i * 128), 128)],
                                out_hbm.at[pl.ds(mo8(out_base + i * 128), 128)])
                return carry
            lax.fori_loop(0, nz, body, 0)
            if rem:
                pltpu.sync_copy(acc.at[pl.ds(mo8(src_base + nz * 128), rem)],
                                out_hbm.at[pl.ds(mo8(out_base + nz * 128), rem)])

        fill_cbuf(0.0)
        for jj in range(_NSH // 2):
            j_val = c * (_NSH // 2) + jj
            zero_acc()
            plsc.subcore_barrier()
            edge_pass(j_val, True)
            plsc.subcore_barrier()
            copy_out(j_val)
            plsc.subcore_barrier()

        if with_counts:
            @pl.when(c == 1)
            def _counts():
                zero_acc()
                plsc.subcore_barrier()
                fill_cbuf(1.0)
                edge_pass(jnp.int32(0), False)
                plsc.subcore_barrier()
                copy_out(jnp.int32(_NSH))
                plsc.subcore_barrier()

    return k(x4, src1, dst2)


def _prep_edges(ei, n_src, n_dst):
    e = ei.shape[1]
    e_pad = _pad_to(e, _TILES * _EB)
    npad = e_pad - e
    ar = jnp.arange(npad, dtype=jnp.int32)
    src = jnp.concatenate([ei[0], ar % n_src])
    dst = jnp.concatenate([ei[1], n_dst + (ar % _DUMP)])
    return src, dst.reshape(e_pad // 128, 128)


def _to4(x):
    n = x.shape[0]
    return x.reshape(n, _NSH, _S).transpose(1, 0, 2).reshape(_NSH * n, _S)


def _from4(out, n_dstp, n_dst):
    s4 = out[: _NSH * n_dstp].reshape(_NSH, n_dstp, _S)[:, :n_dst, :]
    return s4.transpose(1, 0, 2).reshape(n_dst, _F)


def _proj_tc(x, W, b):
    n, f_in = x.shape
    bn = 400
    def kfn(x_ref, w_ref, b_ref, o_ref):
        y = jnp.dot(x_ref[...], w_ref[...], preferred_element_type=jnp.float32)
        o_ref[...] = jnp.maximum(y + b_ref[...], 0.0)
    return pl.pallas_call(
        kfn,
        grid=(n // bn,),
        in_specs=[pl.BlockSpec((bn, f_in), lambda i: (i, 0)),
                  pl.BlockSpec((f_in, _F), lambda i: (0, 0)),
                  pl.BlockSpec((1, _F), lambda i: (0, 0))],
        out_specs=pl.BlockSpec((bn, _F), lambda i: (i, 0)),
        out_shape=jax.ShapeDtypeStruct((n, _F), jnp.float32),
    )(x, W, b.reshape(1, _F))


def _combine_p_tc(s1, c1, s2, c2, x, Wl1, Wl2, Wr1, Wr2, bl1, bl2, relu):
    n = x.shape[0]
    bn = 400
    def kfn(s1_ref, c1_ref, s2_ref, c2_ref, x_ref, wl1, wl2, wr1, wr2,
            b1, b2, o_ref):
        r1 = 1.0 / jnp.maximum(c1_ref[...], 1.0)
        r2 = 1.0 / jnp.maximum(c2_ref[...], 1.0)
        acc = jnp.dot(s1_ref[...] * r1, wl1[...],
                      preferred_element_type=jnp.float32)
        acc = acc + jnp.dot(s2_ref[...] * r2, wl2[...],
                            preferred_element_type=jnp.float32)
        acc = acc + jnp.dot(x_ref[...], wr1[...] + wr2[...],
                            preferred_element_type=jnp.float32)
        acc = (acc + b1[...] + b2[...]) * 0.5
        o_ref[...] = jnp.maximum(acc, 0.0) if relu else acc
    full = lambda i: (0, 0)
    blk = lambda i: (i, 0)
    return pl.pallas_call(
        kfn,
        grid=(n // bn,),
        in_specs=[pl.BlockSpec((bn, _F), blk), pl.BlockSpec((bn, 1), blk),
                  pl.BlockSpec((bn, _F), blk), pl.BlockSpec((bn, 1), blk),
                  pl.BlockSpec((bn, _F), blk),
                  pl.BlockSpec((_F, _F), full), pl.BlockSpec((_F, _F), full),
                  pl.BlockSpec((_F, _F), full), pl.BlockSpec((_F, _F), full),
                  pl.BlockSpec((1, _F), full), pl.BlockSpec((1, _F), full)],
        out_specs=pl.BlockSpec((bn, _F), blk),
        out_shape=jax.ShapeDtypeStruct((n, _F), jnp.float32),
    )(s1, c1, s2, c2, x, Wl1, Wl2, Wr1, Wr2,
      bl1.reshape(1, _F), bl2.reshape(1, _F))


def _combine_t_tc(s1, c1, x, Wl1, Wr1, bl1, relu):
    n = x.shape[0]
    bn = 400
    def kfn(s1_ref, c1_ref, x_ref, wl1, wr1, b1, o_ref):
        r1 = 1.0 / jnp.maximum(c1_ref[...], 1.0)
        acc = jnp.dot(s1_ref[...] * r1, wl1[...],
                      preferred_element_type=jnp.float32)
        acc = acc + jnp.dot(x_ref[...], wr1[...],
                            preferred_element_type=jnp.float32)
        acc = acc + b1[...]
        o_ref[...] = jnp.maximum(acc, 0.0) if relu else acc
    full = lambda i: (0, 0)
    blk = lambda i: (i, 0)
    return pl.pallas_call(
        kfn,
        grid=(n // bn,),
        in_specs=[pl.BlockSpec((bn, _F), blk), pl.BlockSpec((bn, 1), blk),
                  pl.BlockSpec((bn, _F), blk),
                  pl.BlockSpec((_F, _F), full), pl.BlockSpec((_F, _F), full),
                  pl.BlockSpec((1, _F), full)],
        out_specs=pl.BlockSpec((bn, _F), blk),
        out_shape=jax.ShapeDtypeStruct((n, _F), jnp.float32),
    )(s1, c1, x, Wl1, Wr1, bl1.reshape(1, _F))


def kernel(x_protein, x_term, edge_index_pp, edge_index_pt, edge_index_tp,
           W_in_p, b_in_p, W_in_t, b_in_t,
           Wl0_pp, bl0_pp, Wr0_pp,
           Wl0_pt, bl0_pt, Wr0_pt,
           Wl0_tp, bl0_tp, Wr0_tp,
           Wl1_pp, bl1_pp, Wr1_pp,
           Wl1_pt, bl1_pt, Wr1_pt,
           Wl1_tp, bl1_tp, Wr1_tp):
    n_pp = _pad_to(_NP + _DUMP, 128)
    n_tp = _pad_to(_NT + _DUMP, 128)

    xp = _proj_tc(x_protein, W_in_p, b_in_p)
    xt = _proj_tc(x_term, W_in_t, b_in_t)

    spp, dpp = _prep_edges(edge_index_pp, _NP, _NP)
    stp, dtp = _prep_edges(edge_index_tp, _NT, _NP)
    spt, dpt = _prep_edges(edge_index_pt, _NP, _NT)

    Wl = {0: (Wl0_pp, Wl0_tp, Wl0_pt), 1: (Wl1_pp, Wl1_tp, Wl1_pt)}
    Wr = {0: (Wr0_pp, Wr0_tp, Wr0_pt), 1: (Wr1_pp, Wr1_tp, Wr1_pt)}
    bl = {0: (bl0_pp, bl0_tp, bl0_pt), 1: (bl1_pp, bl1_tp, bl1_pt)}

    cnt_pp = cnt_tp = cnt_pt = None
    for l in range(2):
        xp4 = _to4(xp)
        xt4 = _to4(xt)
        with_counts = l == 0
        o_pp = _seg_sum_sc(xp4, spp, dpp, _NP, n_pp, with_counts)
        o_tp = _seg_sum_sc(xt4, stp, dtp, _NT, n_pp, with_counts)
        o_pt = _seg_sum_sc(xp4, spt, dpt, _NP, n_tp, with_counts)
        if with_counts:
            cnt_pp = o_pp[_NSH * n_pp: _NSH * n_pp + _NP, 0:1]
            cnt_tp = o_tp[_NSH * n_pp: _NSH * n_pp + _NP, 0:1]
            cnt_pt = o_pt[_NSH * n_tp: _NSH * n_tp + _NT, 0:1]
        s_pp = _from4(o_pp, n_pp, _NP)
        s_tp = _from4(o_tp, n_pp, _NP)
        s_pt = _from4(o_pt, n_tp, _NT)
        wl_pp, wl_tp, wl_pt = Wl[l]
        wr_pp, wr_tp, wr_pt = Wr[l]
        bl_pp, bl_tp, bl_pt = bl[l]
        xp_new = _combine_p_tc(s_pp, cnt_pp, s_tp, cnt_tp, xp,
                               wl_pp, wl_tp, wr_pp, wr_tp, bl_pp, bl_tp,
                               relu=(l == 0))
        xt_new = _combine_t_tc(s_pt, cnt_pt, xt, wl_pt, wr_pt, bl_pt,
                               relu=(l == 0))
        xp, xt = xp_new, xt_new
    return xp, xt
